# baseline (speedup 1.0000x reference)
import functools

import jax
import jax.numpy as jnp
from jax import lax
from jax.experimental import pallas as pl
from jax.experimental.pallas import tpu as pltpu
from jax.experimental.pallas import tpu_sc as plsc

_CHUNK = 128
_GRP = 8


def _stage_body(tT_ref, o_ref, *, d):
    tT = tT_ref[...]
    cols = tT.reshape(d, -1, _GRP)
    outs = []
    for r in range(_GRP):
        outs.append(jnp.transpose(cols[:, :, r]))
    o_ref[...] = jnp.concatenate(outs, axis=1)


def _stage(tbl_T, d, n):
    w = 1024
    ng = pl.cdiv(n, _GRP)
    return pl.pallas_call(
        functools.partial(_stage_body, d=d),
        grid=(pl.cdiv(n, w),),
        in_specs=[pl.BlockSpec((d, w), lambda m: (0, m))],
        out_specs=pl.BlockSpec((w // _GRP, _GRP * d), lambda m: (m, 0)),
        out_shape=jax.ShapeDtypeStruct((ng, _GRP * d), jnp.float32),
    )(tbl_T)


def _sc_gather(users_r, items_r, uemb_g, iemb_g, ulin_flat, ilin_flat,
               nw, ch):
    nc = plsc.get_sparse_core_info().num_cores

    @functools.partial(
        pl.kernel,
        mesh=plsc.VectorSubcoreMesh(core_axis_name="c", subcore_axis_name="s"),
        out_type=[
            jax.ShapeDtypeStruct((nw * ch * _CHUNK, 128), jnp.float32),
            jax.ShapeDtypeStruct((nw * ch * _CHUNK, 128), jnp.float32),
            jax.ShapeDtypeStruct((nw * ch, _CHUNK), jnp.float32),
            jax.ShapeDtypeStruct((nw * ch, _CHUNK), jnp.float32),
        ],
        scratch_types=[
            pltpu.VMEM((ch, _CHUNK), jnp.int32),
            pltpu.VMEM((ch, _CHUNK), jnp.int32),
            pltpu.VMEM((ch, _CHUNK), jnp.int32),
            pltpu.VMEM((ch, _CHUNK), jnp.int32),
            pltpu.VMEM((2, _CHUNK, 128), jnp.float32),
            pltpu.VMEM((2, _CHUNK, 128), jnp.float32),
            pltpu.VMEM((ch, _CHUNK), jnp.float32),
            pltpu.VMEM((ch, _CHUNK), jnp.float32),
            pltpu.SemaphoreType.DMA,
        ],
    )
    def k(users_h, items_h, uemb_h, iemb_h, ulin_h, ilin_h,
          uout_h, iout_h, ulout_h, ilout_h,
          uidx, iidx, ugidx, igidx, urows, irows, ulv, ilv, sem):
        wid = lax.axis_index("s") * nc + lax.axis_index("c")
        r0 = wid * ch
        pltpu.sync_copy(users_h.at[pl.ds(r0, ch)], uidx)
        pltpu.sync_copy(items_h.at[pl.ds(r0, ch)], iidx)
        for c in range(ch):
            for k16 in range(_CHUNK // 16):
                s = pl.ds(k16 * 16, 16)
                ugidx.at[c][s] = lax.shift_right_logical(uidx.at[c][s], 3)
                igidx.at[c][s] = lax.shift_right_logical(iidx.at[c][s], 3)
        lin_cps = []
        for c in range(ch):
            lin_cps.append(
                pltpu.async_copy(ulin_h.at[uidx.at[c]], ulv.at[c], sem))
            lin_cps.append(
                pltpu.async_copy(ilin_h.at[iidx.at[c]], ilv.at[c], sem))
        cps = [None] * ch
        for c in range(ch):
            b = c % 2
            cps[c] = (
                pltpu.async_copy(uemb_h.at[ugidx.at[c]], urows.at[b], sem),
                pltpu.async_copy(iemb_h.at[igidx.at[c]], irows.at[b], sem),
            )
            if c > 0:
                pc, pb = c - 1, (c - 1) % 2
                cps[pc][0].wait()
                cps[pc][1].wait()
                dst = pl.ds((r0 + pc) * _CHUNK, _CHUNK)
                pltpu.sync_copy(urows.at[pb], uout_h.at[dst])
                pltpu.sync_copy(irows.at[pb], iout_h.at[dst])
        lc, lb = ch - 1, (ch - 1) % 2
        cps[lc][0].wait()
        cps[lc][1].wait()
        dst = pl.ds((r0 + lc) * _CHUNK, _CHUNK)
        pltpu.sync_copy(urows.at[lb], uout_h.at[dst])
        pltpu.sync_copy(irows.at[lb], iout_h.at[dst])
        for cp in lin_cps:
            cp.wait()
        pltpu.sync_copy(ulv, ulout_h.at[pl.ds(r0, ch)])
        pltpu.sync_copy(ilv, ilout_h.at[pl.ds(r0, ch)])

    return k(users_r, items_r, uemb_g, iemb_g, ulin_flat, ilin_flat)


def _tc_body(ug_ref, ig_ref, ul_ref, il_ref, uu_ref, ii_ref,
             w0u_ref, w0i_ref, w1_ref, w2_ref, b0_ref, b1_ref, cb_ref,
             o_ref, *, d):
    bm = ug_ref.shape[0]
    lane_grp = jax.lax.broadcasted_iota(jnp.int32, (1, _GRP * d), 1) // d
    ru = jnp.bitwise_and(uu_ref[...], _GRP - 1)
    ri = jnp.bitwise_and(ii_ref[...], _GRP - 1)
    um = jnp.where(lane_grp == ru, ug_ref[...], 0.0)
    im = jnp.where(lane_grp == ri, ig_ref[...], 0.0)
    s = jnp.sum(um, axis=1, keepdims=True) + jnp.sum(im, axis=1, keepdims=True)
    sq = jnp.sum(um * um, axis=1, keepdims=True) + jnp.sum(im * im, axis=1, keepdims=True)
    fm = 0.5 * (s * s - sq)
    h = (jnp.dot(um, w0u_ref[...], preferred_element_type=jnp.float32)
         + jnp.dot(im, w0i_ref[...], preferred_element_type=jnp.float32)
         + b0_ref[...])
    h = jnp.maximum(h, 0.0)
    h = jnp.maximum(
        jnp.dot(h, w1_ref[...], preferred_element_type=jnp.float32) + b1_ref[...],
        0.0)
    y = jnp.sum(h * w2_ref[...], axis=1, keepdims=True)
    o_ref[...] = ul_ref[...] + il_ref[...] + fm + y + cb_ref[...]


def _tc_mlp(u_g, i_g, ul, il, uu, ii, w0u, w0i, b0r, w1, b1r, w2r, cb, d):
    b = u_g.shape[0]
    bm = 2048
    grid = b // bm
    h0 = w1.shape[0]
    h1 = w1.shape[1]
    return pl.pallas_call(
        functools.partial(_tc_body, d=d),
        grid=(grid,),
        in_specs=[
            pl.BlockSpec((bm, _GRP * d), lambda m: (m, 0)),
            pl.BlockSpec((bm, _GRP * d), lambda m: (m, 0)),
            pl.BlockSpec((bm, 1), lambda m: (m, 0)),
            pl.BlockSpec((bm, 1), lambda m: (m, 0)),
            pl.BlockSpec((bm, 1), lambda m: (m, 0)),
            pl.BlockSpec((bm, 1), lambda m: (m, 0)),
            pl.BlockSpec((_GRP * d, h0), lambda m: (0, 0)),
            pl.BlockSpec((_GRP * d, h0), lambda m: (0, 0)),
            pl.BlockSpec((h0, h1), lambda m: (0, 0)),
            pl.BlockSpec((1, h1), lambda m: (0, 0)),
            pl.BlockSpec((1, h0), lambda m: (0, 0)),
            pl.BlockSpec((1, h1), lambda m: (0, 0)),
            pl.BlockSpec((1, 1), lambda m: (0, 0)),
        ],
        out_specs=pl.BlockSpec((bm, 1), lambda m: (m, 0)),
        out_shape=jax.ShapeDtypeStruct((b, 1), jnp.float32),
    )(u_g, i_g, ul, il, uu, ii, w0u, w0i, w1, w2r, b0r, b1r, cb)


@jax.jit
def kernel(users, items, user_emb, item_emb, user_lin_w, user_lin_b,
           item_lin_w, item_lin_b, W0, b0, W1, b1, W2, b2):
    b = users.shape[0]
    n, d = user_emb.shape
    nw = 32
    ch = b // (nw * _CHUNK)
    users32 = users.astype(jnp.int32)
    items32 = items.astype(jnp.int32)
    users_r = users32.reshape(nw * ch, _CHUNK)
    items_r = items32.reshape(nw * ch, _CHUNK)
    uemb_g = _stage(user_emb.T, d, n)
    iemb_g = _stage(item_emb.T, d, n)
    u_grp, i_grp, u_lin, i_lin = _sc_gather(
        users_r, items_r, uemb_g, iemb_g,
        user_lin_w.reshape(-1), item_lin_w.reshape(-1), nw, ch)
    ul = u_lin.reshape(b, 1)
    il = i_lin.reshape(b, 1)
    cb = (user_lin_b[0] + item_lin_b[0] + b2[0]).reshape(1, 1)
    w0u = jnp.tile(W0[:d], (_GRP, 1))
    w0i = jnp.tile(W0[d:], (_GRP, 1))
    return _tc_mlp(u_grp, i_grp, ul, il,
                   users32.reshape(b, 1), items32.reshape(b, 1),
                   w0u, w0i, b0.reshape(1, -1), W1, b1.reshape(1, -1),
                   W2.reshape(1, -1), cb, d)


# speedup vs baseline: 7.1251x; 7.1251x over previous
import functools

import jax
import jax.numpy as jnp
from jax import lax
from jax.experimental import pallas as pl
from jax.experimental.pallas import tpu as pltpu
from jax.experimental.pallas import tpu_sc as plsc

_CHUNK = 128


def _sc_gather(users_r, items_r, user_emb, item_emb, ulin_flat, ilin_flat,
               nw, ch, d):
    nc = plsc.get_sparse_core_info().num_cores

    @functools.partial(
        pl.kernel,
        mesh=plsc.VectorSubcoreMesh(core_axis_name="c", subcore_axis_name="s"),
        compiler_params=pltpu.CompilerParams(use_tc_tiling_on_sc=False),
        out_type=[
            jax.ShapeDtypeStruct((nw * ch * _CHUNK, d), jnp.float32),
            jax.ShapeDtypeStruct((nw * ch * _CHUNK, d), jnp.float32),
            jax.ShapeDtypeStruct((nw * ch, _CHUNK), jnp.float32),
            jax.ShapeDtypeStruct((nw * ch, _CHUNK), jnp.float32),
        ],
        scratch_types=[
            pltpu.VMEM((ch, _CHUNK), jnp.int32),
            pltpu.VMEM((ch, _CHUNK), jnp.int32),
            pltpu.VMEM((ch, _CHUNK, 16), jnp.float32),
            pltpu.VMEM((ch, _CHUNK, 16), jnp.float32),
            pltpu.VMEM((ch, _CHUNK), jnp.float32),
            pltpu.VMEM((ch, _CHUNK), jnp.float32),
            pltpu.SemaphoreType.DMA,
        ],
    )
    def k(users_h, items_h, uemb_h, iemb_h, ulin_h, ilin_h,
          uout_h, iout_h, ulout_h, ilout_h,
          uidx, iidx, urows, irows, ulv, ilv, sem):
        wid = lax.axis_index("s") * nc + lax.axis_index("c")
        r0 = wid * ch
        pltpu.sync_copy(users_h.at[pl.ds(r0, ch)], uidx)
        pltpu.sync_copy(items_h.at[pl.ds(r0, ch)], iidx)
        cps = []
        for c in range(ch):
            cps.append(pltpu.async_copy(uemb_h.at[uidx.at[c]], urows.at[c], sem))
            cps.append(pltpu.async_copy(iemb_h.at[iidx.at[c]], irows.at[c], sem))
            cps.append(pltpu.async_copy(ulin_h.at[uidx.at[c]], ulv.at[c], sem))
            cps.append(pltpu.async_copy(ilin_h.at[iidx.at[c]], ilv.at[c], sem))
        for cp in cps:
            cp.wait()
        for c in range(ch):
            dst = pl.ds((r0 + c) * _CHUNK, _CHUNK)
            pltpu.sync_copy(urows.at[c], uout_h.at[dst])
            pltpu.sync_copy(irows.at[c], iout_h.at[dst])
        pltpu.sync_copy(ulv, ulout_h.at[pl.ds(r0, ch)])
        pltpu.sync_copy(ilv, ilout_h.at[pl.ds(r0, ch)])

    return k(users_r, items_r, user_emb, item_emb, ulin_flat, ilin_flat)


def _tc_body(u_ref, i_ref, ul_ref, il_ref, w0_ref, w1_ref, w2_ref,
             b0_ref, b1_ref, cb_ref, o_ref, *, d):
    u = u_ref[...]
    it = i_ref[...]
    s = jnp.sum(u, axis=1, keepdims=True) + jnp.sum(it, axis=1, keepdims=True)
    sq = jnp.sum(u * u, axis=1, keepdims=True) + jnp.sum(it * it, axis=1, keepdims=True)
    fm = 0.5 * (s * s - sq)
    w0 = w0_ref[...]
    h = (jnp.dot(u, w0[:d, :], preferred_element_type=jnp.float32)
         + jnp.dot(it, w0[d:, :], preferred_element_type=jnp.float32)
         + b0_ref[...])
    h = jnp.maximum(h, 0.0)
    h = jnp.maximum(
        jnp.dot(h, w1_ref[...], preferred_element_type=jnp.float32) + b1_ref[...],
        0.0)
    y = jnp.sum(h * w2_ref[...], axis=1, keepdims=True)
    o_ref[...] = ul_ref[...] + il_ref[...] + fm + y + cb_ref[...]


def _tc_mlp(u_e, i_e, ul, il, w0, b0r, w1, b1r, w2r, cb):
    b, d = u_e.shape
    bm = 2048
    grid = b // bm
    h0 = w0.shape[1]
    h1 = w1.shape[1]
    return pl.pallas_call(
        functools.partial(_tc_body, d=d),
        grid=(grid,),
        in_specs=[
            pl.BlockSpec((bm, d), lambda m: (m, 0)),
            pl.BlockSpec((bm, d), lambda m: (m, 0)),
            pl.BlockSpec((bm, 1), lambda m: (m, 0)),
            pl.BlockSpec((bm, 1), lambda m: (m, 0)),
            pl.BlockSpec((2 * d, h0), lambda m: (0, 0)),
            pl.BlockSpec((h0, h1), lambda m: (0, 0)),
            pl.BlockSpec((1, h1), lambda m: (0, 0)),
            pl.BlockSpec((1, h0), lambda m: (0, 0)),
            pl.BlockSpec((1, h1), lambda m: (0, 0)),
            pl.BlockSpec((1, 1), lambda m: (0, 0)),
        ],
        out_specs=pl.BlockSpec((bm, 1), lambda m: (m, 0)),
        out_shape=jax.ShapeDtypeStruct((b, 1), jnp.float32),
    )(u_e, i_e, ul, il, w0, w1, w2r, b0r, b1r, cb)


@jax.jit
def kernel(users, items, user_emb, item_emb, user_lin_w, user_lin_b,
           item_lin_w, item_lin_b, W0, b0, W1, b1, W2, b2):
    b = users.shape[0]
    d = user_emb.shape[1]
    nw = 32
    ch = b // (nw * _CHUNK)
    users_r = users.astype(jnp.int32).reshape(nw * ch, _CHUNK)
    items_r = items.astype(jnp.int32).reshape(nw * ch, _CHUNK)
    u_e, i_e, u_lin, i_lin = _sc_gather(
        users_r, items_r, user_emb, item_emb,
        user_lin_w.reshape(-1), item_lin_w.reshape(-1), nw, ch, d)
    ul = u_lin.reshape(b, 1)
    il = i_lin.reshape(b, 1)
    cb = (user_lin_b[0] + item_lin_b[0] + b2[0]).reshape(1, 1)
    return _tc_mlp(u_e, i_e, ul, il, W0, b0.reshape(1, -1), W1,
                   b1.reshape(1, -1), W2.reshape(1, -1), cb)
